# baseline (speedup 1.0000x reference)
import jax
import jax.numpy as jnp
from jax import lax
from jax.experimental import pallas as pl
from jax.experimental.pallas import tpu as pltpu

_BLK = 256
_KB = 0


def _perm(i):
    return jnp.where(i < 2 * _KB,
                     jnp.where(i % 2 == 1, i // 2, _KB + i // 2),
                     i)


def _gcn_body(adj0_ref, adj1_ref, x_ref, aw_ref, ab_ref, w1_ref, b1_ref,
              wm_ref, bm_ref, w2_ref, b2_ref, out_ref,
              adj_vmem, h_ref, s_ref, s3_ref, nz0_ref, nz1_ref):
    p = pl.program_id(0)
    i = pl.program_id(1)
    pi = _perm(i)
    is_res = (i < 2 * _KB) & (i % 2 == 1)
    roff = pi * _BLK

    @pl.when(p == 0)
    def _phase0():
        a0 = adj0_ref[...]
        a1 = adj1_ref[...]
        w = aw_ref[...]
        s3_ref[pl.ds(i * _BLK, _BLK), 0:1] = jnp.sum(
            a0 * w, axis=1, keepdims=True)
        s3_ref[pl.ds(i * _BLK, _BLK), 1:2] = jnp.sum(
            a1 * w, axis=1, keepdims=True)

        @pl.when(i < _KB)
        def _():
            adj_vmem[pl.ds(i * _BLK, _BLK), :] = a0

    @pl.when(p == 1)
    def _phase1():
        @pl.when(i == 0)
        def _():
            one = jnp.ones((1, 1), jnp.float32)
            dn_t = (((1,), (1,)), ((), ()))
            z1 = lax.dot_general(one, s3_ref[:, 0:1], dn_t,
                                 preferred_element_type=jnp.float32)
            z2 = lax.dot_general(one, s3_ref[:, 1:2], dn_t,
                                 preferred_element_type=jnp.float32)
            z1 = z1 + ab_ref[...]
            z2 = z2 + ab_ref[...]
            m = jnp.maximum(z1, z2)
            e1 = jnp.exp(z1 - m)
            e2 = jnp.exp(z2 - m)
            den = e1 + e2
            nz0_ref[...] = e1 / den
            nz1_ref[...] = e2 / den
            s_ref[...] = jnp.dot(x_ref[...], w1_ref[...],
                                 preferred_element_type=jnp.float32)

        nz0 = nz0_ref[...]
        nz1 = nz1_ref[...]
        a1 = adj1_ref[...]
        s1 = s_ref[...]
        b1 = b1_ref[...]

        @pl.when(is_res)
        def _():
            am = nz0 * adj_vmem[pl.ds(roff, _BLK), :] + nz1 * a1
            adj_vmem[pl.ds(roff, _BLK), :] = am
            h_ref[pl.ds(roff, _BLK), :] = jnp.maximum(
                jnp.dot(am, s1, preferred_element_type=jnp.float32) + b1, 0.0)

        @pl.when(jnp.logical_not(is_res))
        def _():
            am = nz0 * adj0_ref[...] + nz1 * a1
            h_ref[pl.ds(roff, _BLK), :] = jnp.maximum(
                jnp.dot(am, s1, preferred_element_type=jnp.float32) + b1, 0.0)

    @pl.when(p == 2)
    def _phase2():
        @pl.when(i == 0)
        def _():
            s_ref[...] = jnp.dot(h_ref[...], wm_ref[...],
                                 preferred_element_type=jnp.float32)

        s2 = s_ref[...]
        bm = bm_ref[...]

        @pl.when(is_res)
        def _():
            h_ref[pl.ds(roff, _BLK), :] = jnp.maximum(
                jnp.dot(adj_vmem[pl.ds(roff, _BLK), :], s2,
                        preferred_element_type=jnp.float32) + bm, 0.0)

        @pl.when(jnp.logical_not(is_res))
        def _():
            am = nz0_ref[...] * adj0_ref[...] + nz1_ref[...] * adj1_ref[...]
            h_ref[pl.ds(roff, _BLK), :] = jnp.maximum(
                jnp.dot(am, s2, preferred_element_type=jnp.float32) + bm, 0.0)

    @pl.when(p == 3)
    def _phase3():
        @pl.when(i == 0)
        def _():
            s3_ref[...] = jnp.dot(h_ref[...], w2_ref[...],
                                  preferred_element_type=jnp.float32)

        s3 = s3_ref[...]
        b2 = b2_ref[...]

        def _softmax_rows(zz):
            m = jnp.max(zz, axis=1, keepdims=True)
            e = jnp.exp(zz - m)
            return e / jnp.sum(e, axis=1, keepdims=True)

        @pl.when(is_res)
        def _():
            zz = jnp.dot(adj_vmem[pl.ds(roff, _BLK), :], s3,
                         preferred_element_type=jnp.float32) + b2
            out_ref[...] = _softmax_rows(zz)

        @pl.when(jnp.logical_not(is_res))
        def _():
            am = nz0_ref[...] * adj0_ref[...] + nz1_ref[...] * adj1_ref[...]
            zz = jnp.dot(am, s3, preferred_element_type=jnp.float32) + b2
            out_ref[...] = _softmax_rows(zz)


def kernel(adj0, adj1, x, adj_origin, atten_w, atten_b, gcn1_w, gcn1_b,
           gcn_w, gcn_b, gcn2_w, gcn2_b):
    del adj_origin
    n = adj0.shape[0]
    f = x.shape[1]
    h = gcn1_w.shape[1]
    c = gcn2_w.shape[1]
    nblk = n // _BLK

    ab = atten_b.reshape(1, 1).astype(jnp.float32)
    b1 = gcn1_b.reshape(1, h)
    bm = gcn_b.reshape(1, h)
    b2 = gcn2_b.reshape(1, c)

    def stream_blk(i):
        return jnp.where(i < 2 * _KB, _KB + i // 2, i)

    def adj0_map(p, i):
        return (jnp.where(p == 0, i, stream_blk(i)), 0)

    def adj1_map(p, i):
        return (jnp.where(p == 0, i,
                          jnp.where(p == 1, _perm(i), stream_blk(i))), 0)

    def const_map(p, i):
        return (0, 0)

    def out_map(p, i):
        return (jnp.where(p == 3, _perm(i), _KB), 0)

    return pl.pallas_call(
        _gcn_body,
        grid=(4, nblk),
        in_specs=[
            pl.BlockSpec((_BLK, n), adj0_map),
            pl.BlockSpec((_BLK, n), adj1_map),
            pl.BlockSpec((n, f), const_map),
            pl.BlockSpec((1, n), const_map),
            pl.BlockSpec((1, 1), const_map),
            pl.BlockSpec((f, h), const_map),
            pl.BlockSpec((1, h), const_map),
            pl.BlockSpec((h, h), const_map),
            pl.BlockSpec((1, h), const_map),
            pl.BlockSpec((h, c), const_map),
            pl.BlockSpec((1, c), const_map),
        ],
        out_specs=pl.BlockSpec((_BLK, c), out_map),
        out_shape=jax.ShapeDtypeStruct((n, c), jnp.float32),
        scratch_shapes=[
            pltpu.VMEM((_KB * _BLK, n), jnp.float32),
            pltpu.VMEM((n, h), jnp.float32),
            pltpu.VMEM((n, h), jnp.float32),
            pltpu.VMEM((n, c), jnp.float32),
            pltpu.VMEM((1, n), jnp.float32),
            pltpu.VMEM((1, n), jnp.float32),
        ],
        compiler_params=pltpu.CompilerParams(
            dimension_semantics=("arbitrary", "arbitrary")),
    )(adj0, adj1, x, atten_w, ab, gcn1_w, b1, gcn_w, bm, gcn2_w, b2)


# speedup vs baseline: 1.1051x; 1.1051x over previous
import jax
import jax.numpy as jnp
from jax import lax
from jax.experimental import pallas as pl
from jax.experimental.pallas import tpu as pltpu

_BLK = 128
_KB = 13
_NBUF = 4


def _outer_body(adj0_hbm, adj1_hbm, x_ref, aw_ref, ab_ref, w1_ref, b1_ref,
                wm_ref, bm_ref, w2_ref, b2_ref, out_ref,
                adj_vmem, h_ref, s_ref, s3_ref, nz0_ref, nz1_ref):
    n = adj0_hbm.shape[0]
    nblk = n // _BLK
    last = nblk - 1

    def inner(idx, adj0_ref, adj1_ref):
        p, i = idx
        roff = i * _BLK

        @pl.when(p == 0)
        def _phase0():
            a0 = adj0_ref[...]
            a1 = adj1_ref[...]
            w = aw_ref[...]
            s3_ref[pl.ds(roff, _BLK), 0:1] = jnp.sum(
                a0 * w, axis=1, keepdims=True)
            s3_ref[pl.ds(roff, _BLK), 1:2] = jnp.sum(
                a1 * w, axis=1, keepdims=True)

            @pl.when(i < _KB)
            def _():
                adj_vmem[pl.ds(roff, _BLK), :] = a0

        @pl.when(p == 1)
        def _phase1():
            @pl.when(i == 0)
            def _():
                one = jnp.ones((1, 1), jnp.float32)
                dn_t = (((1,), (1,)), ((), ()))
                z1 = lax.dot_general(one, s3_ref[:, 0:1], dn_t,
                                     preferred_element_type=jnp.float32)
                z2 = lax.dot_general(one, s3_ref[:, 1:2], dn_t,
                                     preferred_element_type=jnp.float32)
                z1 = z1 + ab_ref[...]
                z2 = z2 + ab_ref[...]
                m = jnp.maximum(z1, z2)
                e1 = jnp.exp(z1 - m)
                e2 = jnp.exp(z2 - m)
                den = e1 + e2
                nz0_ref[...] = e1 / den
                nz1_ref[...] = e2 / den
                s_ref[...] = jnp.dot(x_ref[...], w1_ref[...],
                                     preferred_element_type=jnp.float32)

            nz0 = nz0_ref[...]
            nz1 = nz1_ref[...]
            a1 = adj1_ref[...]
            s1 = s_ref[...]
            b1 = b1_ref[...]

            @pl.when(i < _KB)
            def _():
                am = nz0 * adj_vmem[pl.ds(roff, _BLK), :] + nz1 * a1
                adj_vmem[pl.ds(roff, _BLK), :] = am
                h_ref[pl.ds(roff, _BLK), :] = jnp.maximum(
                    jnp.dot(am, s1, preferred_element_type=jnp.float32)
                    + b1, 0.0)

            @pl.when(i >= _KB)
            def _():
                am = nz0 * adj0_ref[...] + nz1 * a1
                h_ref[pl.ds(roff, _BLK), :] = jnp.maximum(
                    jnp.dot(am, s1, preferred_element_type=jnp.float32)
                    + b1, 0.0)

        @pl.when(p == 2)
        def _phase2():
            @pl.when(i == 0)
            def _():
                s_ref[...] = jnp.dot(h_ref[...], wm_ref[...],
                                     preferred_element_type=jnp.float32)

            s2 = s_ref[...]
            bm = bm_ref[...]

            @pl.when(i < _KB)
            def _():
                h_ref[pl.ds(roff, _BLK), :] = jnp.maximum(
                    jnp.dot(adj_vmem[pl.ds(roff, _BLK), :], s2,
                            preferred_element_type=jnp.float32) + bm, 0.0)

            @pl.when(i >= _KB)
            def _():
                am = (nz0_ref[...] * adj0_ref[...]
                      + nz1_ref[...] * adj1_ref[...])
                h_ref[pl.ds(roff, _BLK), :] = jnp.maximum(
                    jnp.dot(am, s2, preferred_element_type=jnp.float32)
                    + bm, 0.0)

        @pl.when(p == 3)
        def _phase3():
            @pl.when(i == 0)
            def _():
                s3_ref[...] = jnp.dot(h_ref[...], w2_ref[...],
                                      preferred_element_type=jnp.float32)

            s3 = s3_ref[...]
            b2 = b2_ref[...]

            def _softmax_rows(zz):
                m = jnp.max(zz, axis=1, keepdims=True)
                e = jnp.exp(zz - m)
                return e / jnp.sum(e, axis=1, keepdims=True)

            @pl.when(i < _KB)
            def _():
                zz = jnp.dot(adj_vmem[pl.ds(roff, _BLK), :], s3,
                             preferred_element_type=jnp.float32) + b2
                out_ref[pl.ds(roff, _BLK), :] = _softmax_rows(zz)

            @pl.when(i >= _KB)
            def _():
                am = (nz0_ref[...] * adj0_ref[...]
                      + nz1_ref[...] * adj1_ref[...])
                zz = jnp.dot(am, s3, preferred_element_type=jnp.float32) + b2
                out_ref[pl.ds(roff, _BLK), :] = _softmax_rows(zz)

    buf = pl.Buffered(buffer_count=_NBUF, use_lookahead=True)

    def adj0_map(p, i):
        return (jnp.where((p == 0) | (i >= _KB), i, last), 0)

    def adj1_map(p, i):
        return (jnp.where((p <= 1) | (i >= _KB), i, last), 0)

    pipe = pltpu.emit_pipeline(
        inner,
        grid=(4, nblk),
        in_specs=[
            pl.BlockSpec((_BLK, n), adj0_map, pipeline_mode=buf),
            pl.BlockSpec((_BLK, n), adj1_map, pipeline_mode=buf),
        ],
        _explicit_indices=True,
    )
    pipe(adj0_hbm, adj1_hbm)


def kernel(adj0, adj1, x, adj_origin, atten_w, atten_b, gcn1_w, gcn1_b,
           gcn_w, gcn_b, gcn2_w, gcn2_b):
    del adj_origin
    n = adj0.shape[0]
    h = gcn1_w.shape[1]
    c = gcn2_w.shape[1]

    ab = atten_b.reshape(1, 1).astype(jnp.float32)
    b1 = gcn1_b.reshape(1, h)
    bm = gcn_b.reshape(1, h)
    b2 = gcn2_b.reshape(1, c)

    hbm = pl.BlockSpec(memory_space=pltpu.MemorySpace.HBM)
    vmem = pl.BlockSpec(memory_space=pltpu.MemorySpace.VMEM)

    return pl.pallas_call(
        _outer_body,
        in_specs=[hbm, hbm] + [vmem] * 9,
        out_specs=vmem,
        out_shape=jax.ShapeDtypeStruct((n, c), jnp.float32),
        scratch_shapes=[
            pltpu.VMEM((_KB * _BLK, n), jnp.float32),
            pltpu.VMEM((n, h), jnp.float32),
            pltpu.VMEM((n, h), jnp.float32),
            pltpu.VMEM((n, c), jnp.float32),
            pltpu.VMEM((1, n), jnp.float32),
            pltpu.VMEM((1, n), jnp.float32),
        ],
    )(adj0, adj1, x, atten_w, ab, gcn1_w, b1, gcn_w, bm, gcn2_w, b2)
